# baseline (speedup 1.0000x reference)
import functools

import jax
import jax.numpy as jnp
from jax import lax
from jax.experimental import pallas as pl
from jax.experimental.pallas import tpu as pltpu
from jax.experimental.pallas import tpu_sc as plsc

B = 4096
NV = 100001
NF = 64
EMB = 16
NC, NS = 2, 16
NW = NC * NS
BPW = B // NW
BC = 32
NCHUNK = BPW // BC
ROWS_CH = BC * NF
IDX_ROWS = ROWS_CH // 128
BM = 256


def _sc_gather(int_feats, table):
    mesh = plsc.VectorSubcoreMesh(
        core_axis_name="c", subcore_axis_name="s",
        num_cores=NC, num_subcores=NS)

    @functools.partial(
        pl.kernel,
        out_type=jax.ShapeDtypeStruct((B * NF, EMB), jnp.float32),
        mesh=mesh,
        scratch_types=[
            pltpu.VMEM((BC, NF), jnp.int32),
            pltpu.VMEM((IDX_ROWS, 128), jnp.int32),
            pltpu.VMEM((ROWS_CH, EMB), jnp.float32),
            pltpu.SemaphoreType.DMA,
        ],
        compiler_params=pltpu.CompilerParams(use_tc_tiling_on_sc=False),
    )
    def k(feats_hbm, table_hbm, out_hbm, feats_v, idx_v, rows_v, sem):
        wid = lax.axis_index("s") * NC + lax.axis_index("c")
        iota = lax.iota(jnp.int32, 16)
        offs = []
        for c in range(4):
            j = iota + 16 * c
            fid = jnp.where(j < 24, j, jnp.where(j < 44, 24, 25))
            offs.append(fid * NV)

        def chunk_body(ch, carry):
            base = wid * BPW + ch * BC
            pltpu.sync_copy(feats_hbm.at[pl.ds(base, BC)], feats_v)
            for b in range(BC):
                for c in range(4):
                    v = feats_v[b, pl.ds(16 * c, 16)] + offs[c]
                    p = b * NF + 16 * c
                    idx_v[p // 128, pl.ds(p % 128, 16)] = v
            cps = [
                pltpu.async_copy(table_hbm.at[idx_v.at[j]],
                                 rows_v.at[pl.ds(j * 128, 128)], sem)
                for j in range(IDX_ROWS)
            ]
            for cp in cps:
                cp.wait()
            pltpu.sync_copy(rows_v, out_hbm.at[pl.ds(base * NF, ROWS_CH)])
            return carry

        lax.fori_loop(0, NCHUNK, chunk_body, 0)

    return k(int_feats, table)


def _tc_dense(E, int_feats, Ws, Wm, bb, gg, be):
    def body(e_ref, if_ref, ws_ref, wm_ref, b_ref, g_ref, be_ref, o_ref):
        e = e_ref[...]
        h = jnp.dot(e[:, :384], ws_ref[...],
                    preferred_element_type=jnp.float32)
        t = jnp.dot(e[:, 384:], wm_ref[...],
                    preferred_element_type=jnp.float32)
        vals = if_ref[...]
        lane = lax.broadcasted_iota(jnp.int32, vals.shape, 1)
        nzf = jnp.where(vals != 0, 1.0, 0.0)
        c24 = jnp.sum(jnp.where((lane >= 24) & (lane < 44), nzf, 0.0),
                      axis=1, keepdims=True)
        c25 = jnp.sum(jnp.where(lane >= 44, nzf, 0.0),
                      axis=1, keepdims=True)
        r24 = 1.0 / jnp.maximum(c24, 1.0)
        r25 = 1.0 / jnp.maximum(c25, 1.0)
        for g in range(4):
            hg = h[:, 128 * g:128 * (g + 1)]
            if g == 3:
                hg = hg + t[:, :128] * r24 + t[:, 128:] * r25
            hg = hg + b_ref[g][None, :]
            mu = jnp.mean(hg, axis=-1, keepdims=True)
            var = jnp.mean((hg - mu) ** 2, axis=-1, keepdims=True)
            hn = (hg - mu) * lax.rsqrt(var + 1e-5) * g_ref[g][None, :] \
                + be_ref[g][None, :]
            o_ref[:, g, :] = hn * jax.nn.sigmoid(hn)

    return pl.pallas_call(
        body,
        grid=(B // BM,),
        in_specs=[
            pl.BlockSpec((BM, NF * EMB), lambda i: (i, 0)),
            pl.BlockSpec((BM, NF), lambda i: (i, 0)),
            pl.BlockSpec((384, 512), lambda i: (0, 0)),
            pl.BlockSpec((640, 256), lambda i: (0, 0)),
            pl.BlockSpec((4, 128), lambda i: (0, 0)),
            pl.BlockSpec((4, 128), lambda i: (0, 0)),
            pl.BlockSpec((4, 128), lambda i: (0, 0)),
        ],
        out_specs=pl.BlockSpec((BM, 4, 128), lambda i: (i, 0, 0)),
        out_shape=jax.ShapeDtypeStruct((B, 4, 128), jnp.float32),
    )(E, int_feats, Ws, Wm, bb, gg, be)


def kernel(int_feats, tables, W0, b0, g0, be0, W1, b1, g1, be1,
           W2, b2, g2, be2, W3, b3, g3, be3):
    T = tables.reshape(26 * NV, EMB)
    E = _sc_gather(int_feats, T).reshape(B, NF * EMB)

    W3t = W3.T
    Ws = jnp.zeros((384, 512), jnp.float32)
    Ws = Ws.at[0:112, 0:128].set(W0.T)
    Ws = Ws.at[112:224, 128:256].set(W1.T)
    Ws = Ws.at[224:320, 256:384].set(W2.T)
    Ws = Ws.at[320:384, 384:512].set(W3t[:64])
    Wm = jnp.zeros((640, 256), jnp.float32)
    Wm = Wm.at[0:320, 0:128].set(jnp.tile(W3t[64:80], (20, 1)))
    Wm = Wm.at[320:640, 128:256].set(jnp.tile(W3t[80:96], (20, 1)))
    bb = jnp.stack([b0, b1, b2, b3])
    gg = jnp.stack([g0, g1, g2, g3])
    be = jnp.stack([be0, be1, be2, be3])
    return _tc_dense(E, int_feats, Ws, Wm, bb, gg, be)


# speedup vs baseline: 1.0020x; 1.0020x over previous
import functools

import jax
import jax.numpy as jnp
from jax import lax
from jax.experimental import pallas as pl
from jax.experimental.pallas import tpu as pltpu
from jax.experimental.pallas import tpu_sc as plsc

B = 4096
NV = 100001
NF = 64
EMB = 16
NC, NS = 2, 16
NW = NC * NS
BPW = B // NW
BC = 16
NCHUNK = BPW // BC
ROWS_CH = BC * NF
IDX_ROWS = ROWS_CH // 128
DM = 512
BM = 512

_GSTART = [0, 7, 14, 20]
COLMAP = []
for _g, _s in enumerate(_GSTART):
    _n = [7, 7, 6, 4][_g]
    COLMAP += [128 * _g + 16 * (k) for k in range(_n)]
POOL24_COL = 128 * 3 + 16 * 4
POOL25_COL = 128 * 3 + 16 * 5
PAD_COLS = [112, 240, 352, 368, 480, 496]


def _sc_gather_pool(int_feats, table):
    mesh = plsc.VectorSubcoreMesh(
        core_axis_name="c", subcore_axis_name="s",
        num_cores=NC, num_subcores=NS)

    @functools.partial(
        pl.kernel,
        out_type=jax.ShapeDtypeStruct((B, DM), jnp.float32),
        mesh=mesh,
        scratch_types=[
            pltpu.VMEM((BC, NF), jnp.int32),
            pltpu.VMEM((IDX_ROWS, 128), jnp.int32),
            pltpu.VMEM((ROWS_CH, EMB), jnp.float32),
            pltpu.VMEM((BC, DM), jnp.float32),
            pltpu.SemaphoreType.DMA,
        ],
        compiler_params=pltpu.CompilerParams(use_tc_tiling_on_sc=False,
                                             needs_layout_passes=False),
    )
    def k(feats_hbm, table_hbm, out_hbm, feats_v, idx_v, rows_v, out_v, sem):
        wid = lax.axis_index("s") * NC + lax.axis_index("c")
        iota = lax.iota(jnp.int32, 16)
        zvec = jnp.zeros((16,), jnp.float32)
        offs = []
        for c in range(4):
            j = iota + 16 * c
            fid = jnp.where(j < 24, j, jnp.where(j < 44, 24, 25))
            offs.append(fid * NV)
        for b in range(BC):
            for col in PAD_COLS:
                out_v[b, pl.ds(col, 16)] = zvec

        def chunk_body(ch, carry):
            base = wid * BPW + ch * BC
            pltpu.sync_copy(feats_hbm.at[pl.ds(base, BC)], feats_v)
            for b in range(BC):
                for c in range(4):
                    v = feats_v[b, pl.ds(16 * c, 16)] + offs[c]
                    p = b * NF + 16 * c
                    idx_v[p // 128, pl.ds(p % 128, 16)] = v
            cps = [
                pltpu.async_copy(table_hbm.at[idx_v.at[j]],
                                 rows_v.at[pl.ds(j * 128, 128)], sem)
                for j in range(IDX_ROWS)
            ]
            for cp in cps:
                cp.wait()
            for b in range(BC):
                rb = b * NF
                for k24, col in enumerate(COLMAP):
                    out_v[b, pl.ds(col, 16)] = rows_v[rb + k24, :]
                acc24 = rows_v[rb + 24, :]
                for j in range(25, 44):
                    acc24 = acc24 + rows_v[rb + j, :]
                acc25 = rows_v[rb + 44, :]
                for j in range(45, 64):
                    acc25 = acc25 + rows_v[rb + j, :]
                c1 = feats_v[b, pl.ds(16, 16)]
                c2 = feats_v[b, pl.ds(32, 16)]
                c3 = feats_v[b, pl.ds(48, 16)]
                n24 = (plsc.all_reduce_population_count(
                           (c1 != 0) & (iota >= 8))
                       + plsc.all_reduce_population_count(
                           (c2 != 0) & (iota < 12)))
                n25 = (plsc.all_reduce_population_count(
                           (c2 != 0) & (iota >= 12))
                       + plsc.all_reduce_population_count(c3 != 0))
                d24 = jnp.maximum(n24.astype(jnp.float32), 1.0)
                d25 = jnp.maximum(n25.astype(jnp.float32), 1.0)
                out_v[b, pl.ds(POOL24_COL, 16)] = acc24 / d24
                out_v[b, pl.ds(POOL25_COL, 16)] = acc25 / d25
            pltpu.sync_copy(out_v, out_hbm.at[pl.ds(base, BC)])
            return carry

        lax.fori_loop(0, NCHUNK, chunk_body, 0)

    return k(int_feats, table)


def _tc_dense(E, Wc, bb, gg, be):
    def body(e_ref, w_ref, b_ref, g_ref, be_ref, o_ref):
        h = jnp.dot(e_ref[...], w_ref[...],
                    preferred_element_type=jnp.float32)
        for g in range(4):
            hg = h[:, 128 * g:128 * (g + 1)] + b_ref[g][None, :]
            mu = jnp.mean(hg, axis=-1, keepdims=True)
            var = jnp.mean((hg - mu) ** 2, axis=-1, keepdims=True)
            hn = (hg - mu) * lax.rsqrt(var + 1e-5) * g_ref[g][None, :] \
                + be_ref[g][None, :]
            o_ref[:, g, :] = hn * jax.nn.sigmoid(hn)

    return pl.pallas_call(
        body,
        grid=(B // BM,),
        in_specs=[
            pl.BlockSpec((BM, DM), lambda i: (i, 0)),
            pl.BlockSpec((DM, DM), lambda i: (0, 0)),
            pl.BlockSpec((4, 128), lambda i: (0, 0)),
            pl.BlockSpec((4, 128), lambda i: (0, 0)),
            pl.BlockSpec((4, 128), lambda i: (0, 0)),
        ],
        out_specs=pl.BlockSpec((BM, 4, 128), lambda i: (i, 0, 0)),
        out_shape=jax.ShapeDtypeStruct((B, 4, 128), jnp.float32),
    )(E, Wc, bb, gg, be)


def kernel(int_feats, tables, W0, b0, g0, be0, W1, b1, g1, be1,
           W2, b2, g2, be2, W3, b3, g3, be3):
    T = tables.reshape(26 * NV, EMB)
    E = _sc_gather_pool(int_feats, T)

    W3t = W3.T
    Wc = jnp.zeros((DM, DM), jnp.float32)
    Wc = Wc.at[0:112, 0:128].set(W0.T)
    Wc = Wc.at[128:240, 128:256].set(W1.T)
    Wc = Wc.at[256:352, 256:384].set(W2.T)
    Wc = Wc.at[384:448, 384:512].set(W3t[:64])
    Wc = Wc.at[448:464, 384:512].set(W3t[64:80])
    Wc = Wc.at[464:480, 384:512].set(W3t[80:96])
    bb = jnp.stack([b0, b1, b2, b3])
    gg = jnp.stack([g0, g1, g2, g3])
    be = jnp.stack([be0, be1, be2, be3])
    return _tc_dense(E, Wc, bb, gg, be)


# speedup vs baseline: 4.5253x; 4.5161x over previous
import functools

import jax
import jax.numpy as jnp
from jax import lax
from jax.experimental import pallas as pl
from jax.experimental.pallas import tpu as pltpu
from jax.experimental.pallas import tpu_sc as plsc

B = 4096
NV = 100001
VP = 102400
NT = 26
NF = 64
EMB = 16
NC, NS = 2, 16
NW = NC * NS
BPW = B // NW
BC = 16
NCHUNK = BPW // BC
ROWS_CH = BC * NF
IDX_ROWS = ROWS_CH // 128
DM = 512
BM = 512
VB = 4096
NVB = VP // VB

_GSIZE = [7, 7, 6, 4]
COLMAP = []
for _g, _n in enumerate(_GSIZE):
    COLMAP += [128 * _g + 16 * _k for _k in range(_n)]
POOL24_COL = 128 * 3 + 16 * 4
POOL25_COL = 128 * 3 + 16 * 5
PAD_COLS = [112, 240, 352, 368, 480, 496]


def _compact_tables(tables):
    def body(t_ref, o_ref):
        x = t_ref[0]
        o_ref[...] = jnp.concatenate(
            [x[512 * s:512 * (s + 1), :] for s in range(8)], axis=1)

    return pl.pallas_call(
        body,
        grid=(NT, NVB),
        in_specs=[pl.BlockSpec((1, VB, EMB), lambda f, c: (f, c, 0))],
        out_specs=pl.BlockSpec((VB // 8, 128), lambda f, c: (f * NVB + c, 0)),
        out_shape=jax.ShapeDtypeStruct((NT * VP // 8, 128), jnp.float32),
    )(tables)


def _sc_gather_pool(feats2, table):
    mesh = plsc.VectorSubcoreMesh(
        core_axis_name="c", subcore_axis_name="s",
        num_cores=NC, num_subcores=NS)

    @functools.partial(
        pl.kernel,
        out_type=jax.ShapeDtypeStruct((B, DM), jnp.float32),
        mesh=mesh,
        scratch_types=[
            pltpu.VMEM((BC // 2, 128), jnp.int32),
            pltpu.VMEM((IDX_ROWS, 128), jnp.int32),
            pltpu.VMEM((ROWS_CH, EMB), jnp.float32),
            pltpu.VMEM((BC, DM), jnp.float32),
            pltpu.SemaphoreType.DMA,
        ],
        compiler_params=pltpu.CompilerParams(use_tc_tiling_on_sc=False,
                                             needs_layout_passes=False),
    )
    def k(feats_hbm, table_hbm, out_hbm, feats_v, idx_v, rows_v, out_v, sem):
        wid = lax.axis_index("s") * NC + lax.axis_index("c")
        iota = lax.iota(jnp.int32, 16)
        zvec = jnp.zeros((16,), jnp.float32)
        fids = []
        for c in range(4):
            j = iota + 16 * c
            fids.append(jnp.where(j < 24, j, jnp.where(j < 44, 24, 25)))

        def flat_idx(v, c):
            w = v & 4095
            blk = fids[c] * 25 + (v >> 12)
            return (blk << 12) + ((w & 511) << 3) + (w >> 9)
        for b in range(BC):
            for col in PAD_COLS:
                out_v[b, pl.ds(col, 16)] = zvec

        def fv(b, c):
            return feats_v[b // 2, pl.ds((b % 2) * 64 + 16 * c, 16)]

        def chunk_body(ch, carry):
            base = wid * BPW + ch * BC
            pltpu.sync_copy(feats_hbm.at[pl.ds(base // 2, BC // 2)], feats_v)
            for b in range(BC):
                for c in range(4):
                    v = flat_idx(fv(b, c), c)
                    p = b * NF + 16 * c
                    idx_v[p // 128, pl.ds(p % 128, 16)] = v
            cps = [
                pltpu.async_copy(table_hbm.at[idx_v.at[j]],
                                 rows_v.at[pl.ds(j * 128, 128)], sem)
                for j in range(IDX_ROWS)
            ]
            for cp in cps:
                cp.wait()
            for b in range(BC):
                rb = b * NF
                for k24, col in enumerate(COLMAP):
                    out_v[b, pl.ds(col, 16)] = rows_v[rb + k24, :]
                acc24 = rows_v[rb + 24, :]
                for j in range(25, 44):
                    acc24 = acc24 + rows_v[rb + j, :]
                acc25 = rows_v[rb + 44, :]
                for j in range(45, 64):
                    acc25 = acc25 + rows_v[rb + j, :]
                c1 = fv(b, 1)
                c2 = fv(b, 2)
                c3 = fv(b, 3)
                n24 = (plsc.all_reduce_population_count(
                           (c1 != 0) & (iota >= 8))
                       + plsc.all_reduce_population_count(
                           (c2 != 0) & (iota < 12)))
                n25 = (plsc.all_reduce_population_count(
                           (c2 != 0) & (iota >= 12))
                       + plsc.all_reduce_population_count(c3 != 0))
                d24 = jnp.maximum(n24.astype(jnp.float32), 1.0)
                d25 = jnp.maximum(n25.astype(jnp.float32), 1.0)
                out_v[b, pl.ds(POOL24_COL, 16)] = acc24 / d24
                out_v[b, pl.ds(POOL25_COL, 16)] = acc25 / d25
            pltpu.sync_copy(out_v, out_hbm.at[pl.ds(base, BC)])
            return carry

        lax.fori_loop(0, NCHUNK, chunk_body, 0)

    return k(feats2, table)


def _tc_dense(E, Wc, bb, gg, be):
    def body(e_ref, w_ref, b_ref, g_ref, be_ref, o_ref):
        h = jnp.dot(e_ref[...], w_ref[...],
                    preferred_element_type=jnp.float32)
        for g in range(4):
            hg = h[:, 128 * g:128 * (g + 1)] + b_ref[g][None, :]
            mu = jnp.mean(hg, axis=-1, keepdims=True)
            var = jnp.mean((hg - mu) ** 2, axis=-1, keepdims=True)
            hn = (hg - mu) * lax.rsqrt(var + 1e-5) * g_ref[g][None, :] \
                + be_ref[g][None, :]
            o_ref[:, g, :] = hn * jax.nn.sigmoid(hn)

    return pl.pallas_call(
        body,
        grid=(B // BM,),
        in_specs=[
            pl.BlockSpec((BM, DM), lambda i: (i, 0)),
            pl.BlockSpec((DM, DM), lambda i: (0, 0)),
            pl.BlockSpec((4, 128), lambda i: (0, 0)),
            pl.BlockSpec((4, 128), lambda i: (0, 0)),
            pl.BlockSpec((4, 128), lambda i: (0, 0)),
        ],
        out_specs=pl.BlockSpec((BM, 4, 128), lambda i: (i, 0, 0)),
        out_shape=jax.ShapeDtypeStruct((B, 4, 128), jnp.float32),
    )(E, Wc, bb, gg, be)


def kernel(int_feats, tables, W0, b0, g0, be0, W1, b1, g1, be1,
           W2, b2, g2, be2, W3, b3, g3, be3):
    T = _compact_tables(tables).reshape(NT * VP, EMB)
    feats2 = int_feats.reshape(B // 2, 128)
    E = _sc_gather_pool(feats2, T)

    W3t = W3.T
    Wc = jnp.zeros((DM, DM), jnp.float32)
    Wc = Wc.at[0:112, 0:128].set(W0.T)
    Wc = Wc.at[128:240, 128:256].set(W1.T)
    Wc = Wc.at[256:352, 256:384].set(W2.T)
    Wc = Wc.at[384:448, 384:512].set(W3t[:64])
    Wc = Wc.at[448:464, 384:512].set(W3t[64:80])
    Wc = Wc.at[464:480, 384:512].set(W3t[80:96])
    bb = jnp.stack([b0, b1, b2, b3])
    gg = jnp.stack([g0, g1, g2, g3])
    be = jnp.stack([be0, be1, be2, be3])
    return _tc_dense(E, Wc, bb, gg, be)


# speedup vs baseline: 8.2730x; 1.8282x over previous
import functools

import jax
import jax.numpy as jnp
from jax import lax
from jax.experimental import pallas as pl
from jax.experimental.pallas import tpu as pltpu
from jax.experimental.pallas import tpu_sc as plsc

B = 4096
NV = 100001
VP = 102400
NT = 26
NF = 64
EMB = 16
NC, NS = 2, 16
NW = NC * NS
BPW = B // NW
BC = 16
NCHUNK = BPW // BC
ROWS_CH = BC * NF
IDX_ROWS = ROWS_CH // 128
DM = 512
BM = 512
VB = 4096
NVB = VP // VB

_GSIZE = [7, 7, 6, 4]
COLMAP = []
for _g, _n in enumerate(_GSIZE):
    COLMAP += [128 * _g + 16 * _k for _k in range(_n)]
POOL24_COL = 128 * 3 + 16 * 4
POOL25_COL = 128 * 3 + 16 * 5
PAD_COLS = [112, 240, 352, 368, 480, 496]


def _compact_tables(tables_t):
    def body(t_ref, o_ref):
        x = t_ref[0]
        o_ref[...] = jnp.concatenate(
            [x[:, 512 * s:512 * (s + 1)].T for s in range(8)], axis=1)

    return pl.pallas_call(
        body,
        grid=(NT, NVB),
        in_specs=[pl.BlockSpec((1, EMB, VB), lambda f, c: (f, 0, c))],
        out_specs=pl.BlockSpec((VB // 8, 128), lambda f, c: (f * NVB + c, 0)),
        out_shape=jax.ShapeDtypeStruct((NT * VP // 8, 128), jnp.float32),
    )(tables_t)


def _sc_gather_pool(feats2, table):
    mesh = plsc.VectorSubcoreMesh(
        core_axis_name="c", subcore_axis_name="s",
        num_cores=NC, num_subcores=NS)

    @functools.partial(
        pl.kernel,
        out_type=jax.ShapeDtypeStruct((B, DM), jnp.float32),
        mesh=mesh,
        scratch_types=[
            pltpu.VMEM((BC // 2, 128), jnp.int32),
            pltpu.VMEM((IDX_ROWS, 128), jnp.int32),
            pltpu.VMEM((ROWS_CH, EMB), jnp.float32),
            pltpu.VMEM((BC, DM), jnp.float32),
            pltpu.SemaphoreType.DMA,
        ],
        compiler_params=pltpu.CompilerParams(use_tc_tiling_on_sc=False,
                                             needs_layout_passes=False),
    )
    def k(feats_hbm, table_hbm, out_hbm, feats_v, idx_v, rows_v, out_v, sem):
        wid = lax.axis_index("s") * NC + lax.axis_index("c")
        iota = lax.iota(jnp.int32, 16)
        zvec = jnp.zeros((16,), jnp.float32)
        fids = []
        for c in range(4):
            j = iota + 16 * c
            fids.append(jnp.where(j < 24, j, jnp.where(j < 44, 24, 25)))

        def flat_idx(v, c):
            w = v & 4095
            blk = fids[c] * 25 + (v >> 12)
            return (blk << 12) + ((w & 511) << 3) + (w >> 9)
        for b in range(BC):
            for col in PAD_COLS:
                out_v[b, pl.ds(col, 16)] = zvec

        def fv(b, c):
            return feats_v[b // 2, pl.ds((b % 2) * 64 + 16 * c, 16)]

        def chunk_body(ch, carry):
            base = wid * BPW + ch * BC
            pltpu.sync_copy(feats_hbm.at[pl.ds(base // 2, BC // 2)], feats_v)
            for b in range(BC):
                for c in range(4):
                    v = flat_idx(fv(b, c), c)
                    p = b * NF + 16 * c
                    idx_v[p // 128, pl.ds(p % 128, 16)] = v
            cps = [
                pltpu.async_copy(table_hbm.at[idx_v.at[j]],
                                 rows_v.at[pl.ds(j * 128, 128)], sem)
                for j in range(IDX_ROWS)
            ]
            for cp in cps:
                cp.wait()
            for b in range(BC):
                rb = b * NF
                for k24, col in enumerate(COLMAP):
                    out_v[b, pl.ds(col, 16)] = rows_v[rb + k24, :]
                acc24 = rows_v[rb + 24, :]
                for j in range(25, 44):
                    acc24 = acc24 + rows_v[rb + j, :]
                acc25 = rows_v[rb + 44, :]
                for j in range(45, 64):
                    acc25 = acc25 + rows_v[rb + j, :]
                c1 = fv(b, 1)
                c2 = fv(b, 2)
                c3 = fv(b, 3)
                n24 = (plsc.all_reduce_population_count(
                           (c1 != 0) & (iota >= 8))
                       + plsc.all_reduce_population_count(
                           (c2 != 0) & (iota < 12)))
                n25 = (plsc.all_reduce_population_count(
                           (c2 != 0) & (iota >= 12))
                       + plsc.all_reduce_population_count(c3 != 0))
                d24 = jnp.maximum(n24.astype(jnp.float32), 1.0)
                d25 = jnp.maximum(n25.astype(jnp.float32), 1.0)
                out_v[b, pl.ds(POOL24_COL, 16)] = acc24 / d24
                out_v[b, pl.ds(POOL25_COL, 16)] = acc25 / d25
            pltpu.sync_copy(out_v, out_hbm.at[pl.ds(base, BC)])
            return carry

        lax.fori_loop(0, NCHUNK, chunk_body, 0)

    return k(feats2, table)


def _tc_dense(E, Wc, bb, gg, be):
    def body(e_ref, w_ref, b_ref, g_ref, be_ref, o_ref):
        h = jnp.dot(e_ref[...], w_ref[...],
                    preferred_element_type=jnp.float32)
        for g in range(4):
            hg = h[:, 128 * g:128 * (g + 1)] + b_ref[g][None, :]
            mu = jnp.mean(hg, axis=-1, keepdims=True)
            var = jnp.mean((hg - mu) ** 2, axis=-1, keepdims=True)
            hn = (hg - mu) * lax.rsqrt(var + 1e-5) * g_ref[g][None, :] \
                + be_ref[g][None, :]
            o_ref[:, g, :] = hn * jax.nn.sigmoid(hn)

    return pl.pallas_call(
        body,
        grid=(B // BM,),
        in_specs=[
            pl.BlockSpec((BM, DM), lambda i: (i, 0)),
            pl.BlockSpec((DM, DM), lambda i: (0, 0)),
            pl.BlockSpec((4, 128), lambda i: (0, 0)),
            pl.BlockSpec((4, 128), lambda i: (0, 0)),
            pl.BlockSpec((4, 128), lambda i: (0, 0)),
        ],
        out_specs=pl.BlockSpec((BM, 4, 128), lambda i: (i, 0, 0)),
        out_shape=jax.ShapeDtypeStruct((B, 4, 128), jnp.float32),
    )(E, Wc, bb, gg, be)


def kernel(int_feats, tables, W0, b0, g0, be0, W1, b1, g1, be1,
           W2, b2, g2, be2, W3, b3, g3, be3):
    T = _compact_tables(tables.transpose(0, 2, 1)).reshape(NT * VP, EMB)
    feats2 = int_feats.reshape(B // 2, 128)
    E = _sc_gather_pool(feats2, T)

    W3t = W3.T
    Wc = jnp.zeros((DM, DM), jnp.float32)
    Wc = Wc.at[0:112, 0:128].set(W0.T)
    Wc = Wc.at[128:240, 128:256].set(W1.T)
    Wc = Wc.at[256:352, 256:384].set(W2.T)
    Wc = Wc.at[384:448, 384:512].set(W3t[:64])
    Wc = Wc.at[448:464, 384:512].set(W3t[64:80])
    Wc = Wc.at[464:480, 384:512].set(W3t[80:96])
    bb = jnp.stack([b0, b1, b2, b3])
    gg = jnp.stack([g0, g1, g2, g3])
    be = jnp.stack([be0, be1, be2, be3])
    return _tc_dense(E, Wc, bb, gg, be)


# speedup vs baseline: 23.5779x; 2.8500x over previous
import functools

import jax
import jax.numpy as jnp
from jax import lax
from jax.experimental import pallas as pl
from jax.experimental.pallas import tpu as pltpu
from jax.experimental.pallas import tpu_sc as plsc

B = 4096
NV = 100001
VP = 131072
NT = 26
LVB = 14
VBK = 1 << LVB
PW = VBK // 8
NVBK = -(-NV // VBK)
NF = 64
EMB = 16
NC, NS = 2, 16
NW = NC * NS
BPW = B // NW
BC = 16
NCHUNK = BPW // BC
ROWS_CH = BC * NF
IDX_ROWS = ROWS_CH // 128
DM = 512
BM = 512

_GSIZE = [7, 7, 6, 4]
COLMAP = []
for _g, _n in enumerate(_GSIZE):
    COLMAP += [128 * _g + 16 * _k for _k in range(_n)]
POOL24_COL = 128 * 3 + 16 * 4
POOL25_COL = 128 * 3 + 16 * 5
PAD_COLS = [112, 240, 352, 368, 480, 496]


def _compact_tables(tables_t):
    def body(t_ref, i_ref, o_ref):
        x = t_ref[0]
        lim = NV - pl.program_id(1) * VBK
        lane = lax.broadcasted_iota(jnp.int32, (EMB, VBK), 1)
        x = jnp.where(lane < lim, x, 0.0)
        xbig = jnp.concatenate(
            [x[:, PW * s:PW * (s + 1)] for s in range(8)], axis=0)
        o_ref[...] = jax.lax.dot_general(
            xbig, i_ref[...], (((0,), (0,)), ((), ())),
            preferred_element_type=jnp.float32)

    return pl.pallas_call(
        body,
        grid=(NT, NVBK),
        in_specs=[
            pl.BlockSpec((1, EMB, VBK), lambda f, c: (f, 0, c)),
            pl.BlockSpec((128, 128), lambda f, c: (0, 0)),
        ],
        out_specs=pl.BlockSpec(
            (VBK // 8, 128), lambda f, c: (f * (VP // VBK) + c, 0)),
        out_shape=jax.ShapeDtypeStruct((NT * VP // 8, 128), jnp.float32),
    )(tables_t, jnp.eye(128, dtype=jnp.float32))


def _sc_gather_pool(feats2, table):
    mesh = plsc.VectorSubcoreMesh(
        core_axis_name="c", subcore_axis_name="s",
        num_cores=NC, num_subcores=NS)

    @functools.partial(
        pl.kernel,
        out_type=jax.ShapeDtypeStruct((B, DM), jnp.float32),
        mesh=mesh,
        scratch_types=[
            pltpu.VMEM((BC // 2, 128), jnp.int32),
            pltpu.VMEM((IDX_ROWS, 128), jnp.int32),
            pltpu.VMEM((ROWS_CH, EMB), jnp.float32),
            pltpu.VMEM((BC, DM), jnp.float32),
            pltpu.SemaphoreType.DMA,
        ],
        compiler_params=pltpu.CompilerParams(use_tc_tiling_on_sc=False,
                                             needs_layout_passes=False),
    )
    def k(feats_hbm, table_hbm, out_hbm, feats_v, idx_v, rows_v, out_v, sem):
        wid = lax.axis_index("s") * NC + lax.axis_index("c")
        iota = lax.iota(jnp.int32, 16)
        zvec = jnp.zeros((16,), jnp.float32)
        fids = []
        for c in range(4):
            j = iota + 16 * c
            fids.append(jnp.where(j < 24, j, jnp.where(j < 44, 24, 25)))

        def flat_idx(v, c):
            w = v & (VBK - 1)
            return (fids[c] * VP + (v - w)
                    + ((w & (PW - 1)) << 3) + (w >> (LVB - 3)))
        for b in range(BC):
            for col in PAD_COLS:
                out_v[b, pl.ds(col, 16)] = zvec

        def fv(b, c):
            return feats_v[b // 2, pl.ds((b % 2) * 64 + 16 * c, 16)]

        def chunk_body(ch, carry):
            base = wid * BPW + ch * BC
            pltpu.sync_copy(feats_hbm.at[pl.ds(base // 2, BC // 2)], feats_v)
            for b in range(BC):
                for c in range(4):
                    v = flat_idx(fv(b, c), c)
                    p = b * NF + 16 * c
                    idx_v[p // 128, pl.ds(p % 128, 16)] = v
            cps = [
                pltpu.async_copy(table_hbm.at[idx_v.at[j]],
                                 rows_v.at[pl.ds(j * 128, 128)], sem)
                for j in range(IDX_ROWS)
            ]
            for cp in cps:
                cp.wait()
            for b in range(BC):
                rb = b * NF
                for k24, col in enumerate(COLMAP):
                    out_v[b, pl.ds(col, 16)] = rows_v[rb + k24, :]
                acc24 = rows_v[rb + 24, :]
                for j in range(25, 44):
                    acc24 = acc24 + rows_v[rb + j, :]
                acc25 = rows_v[rb + 44, :]
                for j in range(45, 64):
                    acc25 = acc25 + rows_v[rb + j, :]
                c1 = fv(b, 1)
                c2 = fv(b, 2)
                c3 = fv(b, 3)
                n24 = (plsc.all_reduce_population_count(
                           (c1 != 0) & (iota >= 8))
                       + plsc.all_reduce_population_count(
                           (c2 != 0) & (iota < 12)))
                n25 = (plsc.all_reduce_population_count(
                           (c2 != 0) & (iota >= 12))
                       + plsc.all_reduce_population_count(c3 != 0))
                d24 = jnp.maximum(n24.astype(jnp.float32), 1.0)
                d25 = jnp.maximum(n25.astype(jnp.float32), 1.0)
                out_v[b, pl.ds(POOL24_COL, 16)] = acc24 / d24
                out_v[b, pl.ds(POOL25_COL, 16)] = acc25 / d25
            pltpu.sync_copy(out_v, out_hbm.at[pl.ds(base, BC)])
            return carry

        lax.fori_loop(0, NCHUNK, chunk_body, 0)

    return k(feats2, table)


def _tc_dense(E, Wc, bb, gg, be):
    def body(e_ref, w_ref, b_ref, g_ref, be_ref, o_ref):
        h = jnp.dot(e_ref[...], w_ref[...],
                    preferred_element_type=jnp.float32)
        for g in range(4):
            hg = h[:, 128 * g:128 * (g + 1)] + b_ref[g][None, :]
            mu = jnp.mean(hg, axis=-1, keepdims=True)
            var = jnp.mean((hg - mu) ** 2, axis=-1, keepdims=True)
            hn = (hg - mu) * lax.rsqrt(var + 1e-5) * g_ref[g][None, :] \
                + be_ref[g][None, :]
            o_ref[:, g, :] = hn * jax.nn.sigmoid(hn)

    return pl.pallas_call(
        body,
        grid=(B // BM,),
        in_specs=[
            pl.BlockSpec((BM, DM), lambda i: (i, 0)),
            pl.BlockSpec((DM, DM), lambda i: (0, 0)),
            pl.BlockSpec((4, 128), lambda i: (0, 0)),
            pl.BlockSpec((4, 128), lambda i: (0, 0)),
            pl.BlockSpec((4, 128), lambda i: (0, 0)),
        ],
        out_specs=pl.BlockSpec((BM, 4, 128), lambda i: (i, 0, 0)),
        out_shape=jax.ShapeDtypeStruct((B, 4, 128), jnp.float32),
    )(E, Wc, bb, gg, be)


def kernel(int_feats, tables, W0, b0, g0, be0, W1, b1, g1, be1,
           W2, b2, g2, be2, W3, b3, g3, be3):
    T = _compact_tables(tables.transpose(0, 2, 1)).reshape(NT * VP, EMB)
    feats2 = int_feats.reshape(B // 2, 128)
    E = _sc_gather_pool(feats2, T)

    W3t = W3.T
    Wc = jnp.zeros((DM, DM), jnp.float32)
    Wc = Wc.at[0:112, 0:128].set(W0.T)
    Wc = Wc.at[128:240, 128:256].set(W1.T)
    Wc = Wc.at[256:352, 256:384].set(W2.T)
    Wc = Wc.at[384:448, 384:512].set(W3t[:64])
    Wc = Wc.at[448:464, 384:512].set(W3t[64:80])
    Wc = Wc.at[464:480, 384:512].set(W3t[80:96])
    bb = jnp.stack([b0, b1, b2, b3])
    gg = jnp.stack([g0, g1, g2, g3])
    be = jnp.stack([be0, be1, be2, be3])
    return _tc_dense(E, Wc, bb, gg, be)


# speedup vs baseline: 25.6149x; 1.0864x over previous
import functools

import jax
import jax.numpy as jnp
from jax import lax
from jax.experimental import pallas as pl
from jax.experimental.pallas import tpu as pltpu
from jax.experimental.pallas import tpu_sc as plsc

B = 4096
NV = 100001
VP = 131072
NT = 26
LVB = 15
VBK = 1 << LVB
PW = VBK // 8
NVBK = -(-NV // VBK)
NF = 64
EMB = 16
NC, NS = 2, 16
NW = NC * NS
BPW = B // NW
BC = 16
NCHUNK = BPW // BC
ROWS_CH = BC * NF
IDX_ROWS = ROWS_CH // 128
DM = 512
BM = 512

_GSIZE = [7, 7, 6, 4]
COLMAP = []
for _g, _n in enumerate(_GSIZE):
    COLMAP += [128 * _g + 16 * _k for _k in range(_n)]
POOL24_COL = 128 * 3 + 16 * 4
POOL25_COL = 128 * 3 + 16 * 5
PAD_COLS = [112, 240, 352, 368, 480, 496]


def _compact_tables(tables_t):
    def body(t_ref, i_ref, o_ref):
        x = t_ref[0]
        lim = NV - pl.program_id(1) * VBK
        lane = lax.broadcasted_iota(jnp.int32, (EMB, VBK), 1)
        x = jnp.where(lane < lim, x, 0.0)
        xbig = jnp.concatenate(
            [x[:, PW * s:PW * (s + 1)] for s in range(8)], axis=0)
        o_ref[...] = jax.lax.dot_general(
            xbig, i_ref[...], (((0,), (0,)), ((), ())),
            preferred_element_type=jnp.float32)

    return pl.pallas_call(
        body,
        grid=(NT, NVBK),
        in_specs=[
            pl.BlockSpec((1, EMB, VBK), lambda f, c: (f, 0, c)),
            pl.BlockSpec((128, 128), lambda f, c: (0, 0)),
        ],
        out_specs=pl.BlockSpec(
            (VBK // 8, 128), lambda f, c: (f * (VP // VBK) + c, 0)),
        out_shape=jax.ShapeDtypeStruct((NT * VP // 8, 128), jnp.float32),
    )(tables_t, jnp.eye(128, dtype=jnp.float32))


def _sc_gather_pool(feats2, table):
    mesh = plsc.VectorSubcoreMesh(
        core_axis_name="c", subcore_axis_name="s",
        num_cores=NC, num_subcores=NS)

    @functools.partial(
        pl.kernel,
        out_type=jax.ShapeDtypeStruct((B, DM), jnp.float32),
        mesh=mesh,
        scratch_types=[
            pltpu.VMEM((BC // 2, 128), jnp.int32),
            pltpu.VMEM((IDX_ROWS, 128), jnp.int32),
            pltpu.VMEM((ROWS_CH, EMB), jnp.float32),
            pltpu.VMEM((BC, DM), jnp.float32),
            pltpu.SemaphoreType.DMA,
        ],
        compiler_params=pltpu.CompilerParams(use_tc_tiling_on_sc=False,
                                             needs_layout_passes=False),
    )
    def k(feats_hbm, table_hbm, out_hbm, feats_v, idx_v, rows_v, out_v, sem):
        wid = lax.axis_index("s") * NC + lax.axis_index("c")
        iota = lax.iota(jnp.int32, 16)
        zvec = jnp.zeros((16,), jnp.float32)
        fids = []
        for c in range(4):
            j = iota + 16 * c
            fids.append(jnp.where(j < 24, j, jnp.where(j < 44, 24, 25)))

        def flat_idx(v, c):
            w = v & (VBK - 1)
            return (fids[c] * VP + (v - w)
                    + ((w & (PW - 1)) << 3) + (w >> (LVB - 3)))
        for b in range(BC):
            for col in PAD_COLS:
                out_v[b, pl.ds(col, 16)] = zvec

        def fv(b, c):
            return feats_v[b // 2, pl.ds((b % 2) * 64 + 16 * c, 16)]

        def chunk_body(ch, carry):
            base = wid * BPW + ch * BC
            pltpu.sync_copy(feats_hbm.at[pl.ds(base // 2, BC // 2)], feats_v)
            for b in range(BC):
                for c in range(4):
                    v = flat_idx(fv(b, c), c)
                    p = b * NF + 16 * c
                    idx_v[p // 128, pl.ds(p % 128, 16)] = v
            cps = [
                pltpu.async_copy(table_hbm.at[idx_v.at[j]],
                                 rows_v.at[pl.ds(j * 128, 128)], sem)
                for j in range(IDX_ROWS)
            ]
            for cp in cps:
                cp.wait()
            for b in range(BC):
                rb = b * NF
                for k24, col in enumerate(COLMAP):
                    out_v[b, pl.ds(col, 16)] = rows_v[rb + k24, :]
                acc24 = rows_v[rb + 24, :]
                for j in range(25, 44):
                    acc24 = acc24 + rows_v[rb + j, :]
                acc25 = rows_v[rb + 44, :]
                for j in range(45, 64):
                    acc25 = acc25 + rows_v[rb + j, :]
                c1 = fv(b, 1)
                c2 = fv(b, 2)
                c3 = fv(b, 3)
                n24 = (plsc.all_reduce_population_count(
                           (c1 != 0) & (iota >= 8))
                       + plsc.all_reduce_population_count(
                           (c2 != 0) & (iota < 12)))
                n25 = (plsc.all_reduce_population_count(
                           (c2 != 0) & (iota >= 12))
                       + plsc.all_reduce_population_count(c3 != 0))
                d24 = jnp.maximum(n24.astype(jnp.float32), 1.0)
                d25 = jnp.maximum(n25.astype(jnp.float32), 1.0)
                out_v[b, pl.ds(POOL24_COL, 16)] = acc24 / d24
                out_v[b, pl.ds(POOL25_COL, 16)] = acc25 / d25
            pltpu.sync_copy(out_v, out_hbm.at[pl.ds(base, BC)])
            return carry

        lax.fori_loop(0, NCHUNK, chunk_body, 0)

    return k(feats2, table)


def _tc_dense(E, Wc, bb, gg, be):
    def body(e_ref, w_ref, b_ref, g_ref, be_ref, o_ref):
        h = jnp.dot(e_ref[...], w_ref[...],
                    preferred_element_type=jnp.float32)
        for g in range(4):
            hg = h[:, 128 * g:128 * (g + 1)] + b_ref[g][None, :]
            mu = jnp.mean(hg, axis=-1, keepdims=True)
            var = jnp.mean((hg - mu) ** 2, axis=-1, keepdims=True)
            hn = (hg - mu) * lax.rsqrt(var + 1e-5) * g_ref[g][None, :] \
                + be_ref[g][None, :]
            o_ref[:, g, :] = hn * jax.nn.sigmoid(hn)

    return pl.pallas_call(
        body,
        grid=(B // BM,),
        in_specs=[
            pl.BlockSpec((BM, DM), lambda i: (i, 0)),
            pl.BlockSpec((DM, DM), lambda i: (0, 0)),
            pl.BlockSpec((4, 128), lambda i: (0, 0)),
            pl.BlockSpec((4, 128), lambda i: (0, 0)),
            pl.BlockSpec((4, 128), lambda i: (0, 0)),
        ],
        out_specs=pl.BlockSpec((BM, 4, 128), lambda i: (i, 0, 0)),
        out_shape=jax.ShapeDtypeStruct((B, 4, 128), jnp.float32),
    )(E, Wc, bb, gg, be)


def kernel(int_feats, tables, W0, b0, g0, be0, W1, b1, g1, be1,
           W2, b2, g2, be2, W3, b3, g3, be3):
    T = _compact_tables(tables.transpose(0, 2, 1)).reshape(NT * VP, EMB)
    feats2 = int_feats.reshape(B // 2, 128)
    E = _sc_gather_pool(feats2, T)

    W3t = W3.T
    Wc = jnp.zeros((DM, DM), jnp.float32)
    Wc = Wc.at[0:112, 0:128].set(W0.T)
    Wc = Wc.at[128:240, 128:256].set(W1.T)
    Wc = Wc.at[256:352, 256:384].set(W2.T)
    Wc = Wc.at[384:448, 384:512].set(W3t[:64])
    Wc = Wc.at[448:464, 384:512].set(W3t[64:80])
    Wc = Wc.at[464:480, 384:512].set(W3t[80:96])
    bb = jnp.stack([b0, b1, b2, b3])
    gg = jnp.stack([g0, g1, g2, g3])
    be = jnp.stack([be0, be1, be2, be3])
    return _tc_dense(E, Wc, bb, gg, be)


# speedup vs baseline: 30.8224x; 1.2033x over previous
import functools

import jax
import jax.numpy as jnp
from jax import lax
from jax.experimental import pallas as pl
from jax.experimental.pallas import tpu as pltpu
from jax.experimental.pallas import tpu_sc as plsc

B = 4096
NV = 100001
VP = 131072
NT = 26
LVB = 16
VBK = 1 << LVB
PW = VBK // 8
NVBK = -(-NV // VBK)
NF = 64
EMB = 16
NC, NS = 2, 16
NW = NC * NS
BPW = B // NW
BC = 16
NCHUNK = BPW // BC
ROWS_CH = BC * NF
IDX_ROWS = ROWS_CH // 128
DM = 512
BM = 512

_GSIZE = [7, 7, 6, 4]
COLMAP = []
for _g, _n in enumerate(_GSIZE):
    COLMAP += [128 * _g + 16 * _k for _k in range(_n)]
POOL24_COL = 128 * 3 + 16 * 4
POOL25_COL = 128 * 3 + 16 * 5
PAD_COLS = [112, 240, 352, 368, 480, 496]


def _compact_tables(tables_t):
    def body(t_ref, i_ref, o_ref):
        x = t_ref[0]
        lim = NV - pl.program_id(1) * VBK
        lane = lax.broadcasted_iota(jnp.int32, (EMB, VBK), 1)
        x = jnp.where(lane < lim, x, 0.0)
        xbig = jnp.concatenate(
            [x[:, PW * s:PW * (s + 1)] for s in range(8)], axis=0)
        o_ref[...] = jax.lax.dot_general(
            xbig, i_ref[...], (((0,), (0,)), ((), ())),
            preferred_element_type=jnp.float32)

    return pl.pallas_call(
        body,
        grid=(NT, NVBK),
        in_specs=[
            pl.BlockSpec((1, EMB, VBK), lambda f, c: (f, 0, c)),
            pl.BlockSpec((128, 128), lambda f, c: (0, 0)),
        ],
        out_specs=pl.BlockSpec(
            (VBK // 8, 128), lambda f, c: (f * (VP // VBK) + c, 0)),
        out_shape=jax.ShapeDtypeStruct((NT * VP // 8, 128), jnp.float32),
    )(tables_t, jnp.eye(128, dtype=jnp.float32))


def _sc_gather_pool(feats2, table):
    mesh = plsc.VectorSubcoreMesh(
        core_axis_name="c", subcore_axis_name="s",
        num_cores=NC, num_subcores=NS)

    @functools.partial(
        pl.kernel,
        out_type=jax.ShapeDtypeStruct((B, DM), jnp.float32),
        mesh=mesh,
        scratch_types=[
            pltpu.VMEM((BC // 2, 128), jnp.int32),
            pltpu.VMEM((IDX_ROWS, 128), jnp.int32),
            pltpu.VMEM((ROWS_CH, EMB), jnp.float32),
            pltpu.VMEM((BC, DM), jnp.float32),
            pltpu.SemaphoreType.DMA,
        ],
        compiler_params=pltpu.CompilerParams(use_tc_tiling_on_sc=False,
                                             needs_layout_passes=False),
    )
    def k(feats_hbm, table_hbm, out_hbm, feats_v, idx_v, rows_v, out_v, sem):
        wid = lax.axis_index("s") * NC + lax.axis_index("c")
        iota = lax.iota(jnp.int32, 16)
        zvec = jnp.zeros((16,), jnp.float32)
        fids = []
        for c in range(4):
            j = iota + 16 * c
            fids.append(jnp.where(j < 24, j, jnp.where(j < 44, 24, 25)))

        def flat_idx(v, c):
            w = v & (VBK - 1)
            return (fids[c] * VP + (v - w)
                    + ((w & (PW - 1)) << 3) + (w >> (LVB - 3)))
        for b in range(BC):
            for col in PAD_COLS:
                out_v[b, pl.ds(col, 16)] = zvec

        def fv(b, c):
            return feats_v[b // 2, pl.ds((b % 2) * 64 + 16 * c, 16)]

        def chunk_body(ch, carry):
            base = wid * BPW + ch * BC
            pltpu.sync_copy(feats_hbm.at[pl.ds(base // 2, BC // 2)], feats_v)
            for b in range(BC):
                for c in range(4):
                    v = flat_idx(fv(b, c), c)
                    p = b * NF + 16 * c
                    idx_v[p // 128, pl.ds(p % 128, 16)] = v
            cps = [
                pltpu.async_copy(table_hbm.at[idx_v.at[j]],
                                 rows_v.at[pl.ds(j * 128, 128)], sem)
                for j in range(IDX_ROWS)
            ]
            for cp in cps:
                cp.wait()
            for b in range(BC):
                rb = b * NF
                for k24, col in enumerate(COLMAP):
                    out_v[b, pl.ds(col, 16)] = rows_v[rb + k24, :]
                acc24 = rows_v[rb + 24, :]
                for j in range(25, 44):
                    acc24 = acc24 + rows_v[rb + j, :]
                acc25 = rows_v[rb + 44, :]
                for j in range(45, 64):
                    acc25 = acc25 + rows_v[rb + j, :]
                c1 = fv(b, 1)
                c2 = fv(b, 2)
                c3 = fv(b, 3)
                n24 = (plsc.all_reduce_population_count(
                           (c1 != 0) & (iota >= 8))
                       + plsc.all_reduce_population_count(
                           (c2 != 0) & (iota < 12)))
                n25 = (plsc.all_reduce_population_count(
                           (c2 != 0) & (iota >= 12))
                       + plsc.all_reduce_population_count(c3 != 0))
                d24 = jnp.maximum(n24.astype(jnp.float32), 1.0)
                d25 = jnp.maximum(n25.astype(jnp.float32), 1.0)
                out_v[b, pl.ds(POOL24_COL, 16)] = acc24 / d24
                out_v[b, pl.ds(POOL25_COL, 16)] = acc25 / d25
            pltpu.sync_copy(out_v, out_hbm.at[pl.ds(base, BC)])
            return carry

        lax.fori_loop(0, NCHUNK, chunk_body, 0)

    return k(feats2, table)


def _tc_dense(E, Wc, bb, gg, be):
    def body(e_ref, w_ref, b_ref, g_ref, be_ref, o_ref):
        h = jnp.dot(e_ref[...], w_ref[...],
                    preferred_element_type=jnp.float32)
        for g in range(4):
            hg = h[:, 128 * g:128 * (g + 1)] + b_ref[g][None, :]
            mu = jnp.mean(hg, axis=-1, keepdims=True)
            var = jnp.mean((hg - mu) ** 2, axis=-1, keepdims=True)
            hn = (hg - mu) * lax.rsqrt(var + 1e-5) * g_ref[g][None, :] \
                + be_ref[g][None, :]
            o_ref[:, g, :] = hn * jax.nn.sigmoid(hn)

    return pl.pallas_call(
        body,
        grid=(B // BM,),
        in_specs=[
            pl.BlockSpec((BM, DM), lambda i: (i, 0)),
            pl.BlockSpec((DM, DM), lambda i: (0, 0)),
            pl.BlockSpec((4, 128), lambda i: (0, 0)),
            pl.BlockSpec((4, 128), lambda i: (0, 0)),
            pl.BlockSpec((4, 128), lambda i: (0, 0)),
        ],
        out_specs=pl.BlockSpec((BM, 4, 128), lambda i: (i, 0, 0)),
        out_shape=jax.ShapeDtypeStruct((B, 4, 128), jnp.float32),
    )(E, Wc, bb, gg, be)


def kernel(int_feats, tables, W0, b0, g0, be0, W1, b1, g1, be1,
           W2, b2, g2, be2, W3, b3, g3, be3):
    T = _compact_tables(tables.transpose(0, 2, 1)).reshape(NT * VP, EMB)
    feats2 = int_feats.reshape(B // 2, 128)
    E = _sc_gather_pool(feats2, T)

    W3t = W3.T
    Wc = jnp.zeros((DM, DM), jnp.float32)
    Wc = Wc.at[0:112, 0:128].set(W0.T)
    Wc = Wc.at[128:240, 128:256].set(W1.T)
    Wc = Wc.at[256:352, 256:384].set(W2.T)
    Wc = Wc.at[384:448, 384:512].set(W3t[:64])
    Wc = Wc.at[448:464, 384:512].set(W3t[64:80])
    Wc = Wc.at[464:480, 384:512].set(W3t[80:96])
    bb = jnp.stack([b0, b1, b2, b3])
    gg = jnp.stack([g0, g1, g2, g3])
    be = jnp.stack([be0, be1, be2, be3])
    return _tc_dense(E, Wc, bb, gg, be)


# speedup vs baseline: 32.2361x; 1.0459x over previous
import functools

import jax
import jax.numpy as jnp
from jax import lax
from jax.experimental import pallas as pl
from jax.experimental.pallas import tpu as pltpu
from jax.experimental.pallas import tpu_sc as plsc

B = 4096
NV = 100001
VP = 131072
NT = 26
LVB = 17
VBK = 1 << LVB
PW = VBK // 8
NVBK = -(-NV // VBK)
NF = 64
EMB = 16
NC, NS = 2, 16
NW = NC * NS
BPW = B // NW
BC = 16
NCHUNK = BPW // BC
ROWS_CH = BC * NF
IDX_ROWS = ROWS_CH // 128
DM = 512
BM = 512

_GSIZE = [7, 7, 6, 4]
COLMAP = []
for _g, _n in enumerate(_GSIZE):
    COLMAP += [128 * _g + 16 * _k for _k in range(_n)]
POOL24_COL = 128 * 3 + 16 * 4
POOL25_COL = 128 * 3 + 16 * 5
PAD_COLS = [112, 240, 352, 368, 480, 496]


def _compact_tables(tables_t):
    def body(t_ref, i_ref, o_ref):
        x = t_ref[0]
        lim = NV - pl.program_id(1) * VBK
        lane = lax.broadcasted_iota(jnp.int32, (EMB, VBK), 1)
        x = jnp.where(lane < lim, x, 0.0)
        xbig = jnp.concatenate(
            [x[:, PW * s:PW * (s + 1)] for s in range(8)], axis=0)
        o_ref[...] = jax.lax.dot_general(
            xbig, i_ref[...], (((0,), (0,)), ((), ())),
            preferred_element_type=jnp.float32)

    return pl.pallas_call(
        body,
        grid=(NT, NVBK),
        in_specs=[
            pl.BlockSpec((1, EMB, VBK), lambda f, c: (f, 0, c)),
            pl.BlockSpec((128, 128), lambda f, c: (0, 0)),
        ],
        out_specs=pl.BlockSpec(
            (VBK // 8, 128), lambda f, c: (f * (VP // VBK) + c, 0)),
        out_shape=jax.ShapeDtypeStruct((NT * VP // 8, 128), jnp.float32),
    )(tables_t, jnp.eye(128, dtype=jnp.float32))


def _sc_gather_pool(feats2, table):
    mesh = plsc.VectorSubcoreMesh(
        core_axis_name="c", subcore_axis_name="s",
        num_cores=NC, num_subcores=NS)

    @functools.partial(
        pl.kernel,
        out_type=jax.ShapeDtypeStruct((B, DM), jnp.float32),
        mesh=mesh,
        scratch_types=[
            pltpu.VMEM((BC // 2, 128), jnp.int32),
            pltpu.VMEM((IDX_ROWS, 128), jnp.int32),
            pltpu.VMEM((ROWS_CH, EMB), jnp.float32),
            pltpu.VMEM((BC, DM), jnp.float32),
            pltpu.SemaphoreType.DMA,
        ],
        compiler_params=pltpu.CompilerParams(use_tc_tiling_on_sc=False,
                                             needs_layout_passes=False),
    )
    def k(feats_hbm, table_hbm, out_hbm, feats_v, idx_v, rows_v, out_v, sem):
        wid = lax.axis_index("s") * NC + lax.axis_index("c")
        iota = lax.iota(jnp.int32, 16)
        zvec = jnp.zeros((16,), jnp.float32)
        fids = []
        for c in range(4):
            j = iota + 16 * c
            fids.append(jnp.where(j < 24, j, jnp.where(j < 44, 24, 25)))

        def flat_idx(v, c):
            w = v & (VBK - 1)
            return (fids[c] * VP + (v - w)
                    + ((w & (PW - 1)) << 3) + (w >> (LVB - 3)))
        for b in range(BC):
            for col in PAD_COLS:
                out_v[b, pl.ds(col, 16)] = zvec

        def fv(b, c):
            return feats_v[b // 2, pl.ds((b % 2) * 64 + 16 * c, 16)]

        def chunk_body(ch, carry):
            base = wid * BPW + ch * BC
            pltpu.sync_copy(feats_hbm.at[pl.ds(base // 2, BC // 2)], feats_v)
            for b in range(BC):
                for c in range(4):
                    v = flat_idx(fv(b, c), c)
                    p = b * NF + 16 * c
                    idx_v[p // 128, pl.ds(p % 128, 16)] = v
            cps = [
                pltpu.async_copy(table_hbm.at[idx_v.at[j]],
                                 rows_v.at[pl.ds(j * 128, 128)], sem)
                for j in range(IDX_ROWS)
            ]
            for cp in cps:
                cp.wait()
            for b in range(BC):
                rb = b * NF
                for k24, col in enumerate(COLMAP):
                    out_v[b, pl.ds(col, 16)] = rows_v[rb + k24, :]
                acc24 = rows_v[rb + 24, :]
                for j in range(25, 44):
                    acc24 = acc24 + rows_v[rb + j, :]
                acc25 = rows_v[rb + 44, :]
                for j in range(45, 64):
                    acc25 = acc25 + rows_v[rb + j, :]
                c1 = fv(b, 1)
                c2 = fv(b, 2)
                c3 = fv(b, 3)
                n24 = (plsc.all_reduce_population_count(
                           (c1 != 0) & (iota >= 8))
                       + plsc.all_reduce_population_count(
                           (c2 != 0) & (iota < 12)))
                n25 = (plsc.all_reduce_population_count(
                           (c2 != 0) & (iota >= 12))
                       + plsc.all_reduce_population_count(c3 != 0))
                d24 = jnp.maximum(n24.astype(jnp.float32), 1.0)
                d25 = jnp.maximum(n25.astype(jnp.float32), 1.0)
                out_v[b, pl.ds(POOL24_COL, 16)] = acc24 / d24
                out_v[b, pl.ds(POOL25_COL, 16)] = acc25 / d25
            pltpu.sync_copy(out_v, out_hbm.at[pl.ds(base, BC)])
            return carry

        lax.fori_loop(0, NCHUNK, chunk_body, 0)

    return k(feats2, table)


def _tc_dense(E, Wc, bb, gg, be):
    def body(e_ref, w_ref, b_ref, g_ref, be_ref, o_ref):
        h = jnp.dot(e_ref[...], w_ref[...],
                    preferred_element_type=jnp.float32)
        for g in range(4):
            hg = h[:, 128 * g:128 * (g + 1)] + b_ref[g][None, :]
            mu = jnp.mean(hg, axis=-1, keepdims=True)
            var = jnp.mean((hg - mu) ** 2, axis=-1, keepdims=True)
            hn = (hg - mu) * lax.rsqrt(var + 1e-5) * g_ref[g][None, :] \
                + be_ref[g][None, :]
            o_ref[:, g, :] = hn * jax.nn.sigmoid(hn)

    return pl.pallas_call(
        body,
        grid=(B // BM,),
        in_specs=[
            pl.BlockSpec((BM, DM), lambda i: (i, 0)),
            pl.BlockSpec((DM, DM), lambda i: (0, 0)),
            pl.BlockSpec((4, 128), lambda i: (0, 0)),
            pl.BlockSpec((4, 128), lambda i: (0, 0)),
            pl.BlockSpec((4, 128), lambda i: (0, 0)),
        ],
        out_specs=pl.BlockSpec((BM, 4, 128), lambda i: (i, 0, 0)),
        out_shape=jax.ShapeDtypeStruct((B, 4, 128), jnp.float32),
    )(E, Wc, bb, gg, be)


def kernel(int_feats, tables, W0, b0, g0, be0, W1, b1, g1, be1,
           W2, b2, g2, be2, W3, b3, g3, be3):
    T = _compact_tables(tables.transpose(0, 2, 1)).reshape(NT * VP, EMB)
    feats2 = int_feats.reshape(B // 2, 128)
    E = _sc_gather_pool(feats2, T)

    W3t = W3.T
    Wc = jnp.zeros((DM, DM), jnp.float32)
    Wc = Wc.at[0:112, 0:128].set(W0.T)
    Wc = Wc.at[128:240, 128:256].set(W1.T)
    Wc = Wc.at[256:352, 256:384].set(W2.T)
    Wc = Wc.at[384:448, 384:512].set(W3t[:64])
    Wc = Wc.at[448:464, 384:512].set(W3t[64:80])
    Wc = Wc.at[464:480, 384:512].set(W3t[80:96])
    bb = jnp.stack([b0, b1, b2, b3])
    gg = jnp.stack([g0, g1, g2, g3])
    be = jnp.stack([be0, be1, be2, be3])
    return _tc_dense(E, Wc, bb, gg, be)
